# baseline (speedup 1.0000x reference)
import jax
import jax.numpy as jnp
from jax import lax
from jax.experimental import pallas as pl
from jax.experimental.pallas import tpu as pltpu
from jax.experimental.pallas import tpu_sc as plsc

B = 16384
D = 64
NW = 32
BPW = B // NW
L = 16
SG = 4


def _dot_body(uids, iids, uembT, iembT, ubias, ibias, out,
              uidx_v, iidx_v, slabs, bslabs, ucol_v, icol_v, out_v, sem):
    wid = lax.axis_index("s") * 2 + lax.axis_index("c")
    base = wid * BPW

    pltpu.sync_copy(uids.at[pl.ds(base, BPW)], uidx_v)
    pltpu.sync_copy(iids.at[pl.ds(base, BPW)], iidx_v)

    dvec = lax.iota(jnp.int32, L)
    lane = lax.iota(jnp.int32, L)
    GIDS = 64
    NPH = 2 * (GIDS // SG)
    NBUF = 3

    def group(g, carry):
        idus = [uidx_v[pl.ds(g * GIDS + v * L, L)] for v in range(GIDS // L)]
        idis = [iidx_v[pl.ds(g * GIDS + v * L, L)] for v in range(GIDS // L)]

        def issue(p):
            q, ids = divmod(p, 2)
            idvecs = idis if ids else idus
            table = iembT if ids else uembT
            btable = ibias if ids else ubias
            cps, offs = [], []
            for j in range(SG):
                e = q * SG + j
                tid = idvecs[e // L][e % L]
                c0 = pl.multiple_of((tid >> 7) << 7, 128)
                cps.append(pltpu.async_copy(
                    table.at[:, pl.ds(c0, 128)], slabs.at[p % NBUF, j], sem))
                cps.append(pltpu.async_copy(
                    btable.at[pl.ds(c0, 128)], bslabs.at[p % NBUF, j], sem))
                offs.append(tid & 127)
            return cps, offs

        def extract(p, offs, col_v):
            bvs = []
            for j in range(SG):
                o = jnp.full((L,), offs[j], jnp.int32)
                for k in range(D // L):
                    col_v[pl.ds(j * D + k * L, L)] = plsc.load_gather(
                        slabs.at[p % NBUF, j], [dvec + k * L, o])
                bvs.append(plsc.load_gather(bslabs.at[p % NBUF, j], [o]))
            return bvs

        accvs = [jnp.zeros((L,), jnp.float32) for _ in range(GIDS // L)]
        ring = [issue(0), issue(1)]
        ub_pend = None
        for p in range(NPH):
            if p + 2 < NPH:
                ring.append(issue(p + 2))
            cps, offs = ring[p]
            for c in cps:
                c.wait()
            q, ids = divmod(p, 2)
            bvs = extract(p, offs, icol_v if ids else ucol_v)
            if ids:
                for j in range(SG):
                    acc = (ucol_v[pl.ds(j * D, L)] * icol_v[pl.ds(j * D, L)])
                    for k in range(1, D // L):
                        acc = acc + (ucol_v[pl.ds(j * D + k * L, L)]
                                     * icol_v[pl.ds(j * D + k * L, L)])
                    s = jnp.sum(acc) + ub_pend[j][0] + bvs[j][0]
                    e = q * SG + j
                    accvs[e // L] = jnp.where(
                        lane == e % L, s, accvs[e // L])
            else:
                ub_pend = bvs
        for v in range(GIDS // L):
            out_v[pl.ds(g * GIDS + v * L, L)] = accvs[v]
        return carry

    lax.fori_loop(0, BPW // GIDS, group, 0)
    pltpu.sync_copy(out_v, out.at[pl.ds(base, BPW)])


def kernel(user_ids, item_ids, user_embed, item_embed, user_bias, item_bias):
    uids = user_ids.astype(jnp.int32)
    iids = item_ids.astype(jnp.int32)
    ueT = user_embed.T
    ieT = item_embed.T
    ub1 = user_bias.reshape(-1)
    ib1 = item_bias.reshape(-1)

    mesh = plsc.VectorSubcoreMesh(core_axis_name="c", subcore_axis_name="s")
    dot = pl.kernel(
        _dot_body,
        mesh=mesh,
        out_type=jax.ShapeDtypeStruct((B,), jnp.float32),
        compiler_params=pltpu.CompilerParams(
            needs_layout_passes=False, use_tc_tiling_on_sc=True
        ),
        scratch_types=[
            pltpu.VMEM((BPW,), jnp.int32),
            pltpu.VMEM((BPW,), jnp.int32),
            pltpu.VMEM((3, SG, D, 128), jnp.float32),
            pltpu.VMEM((3, SG, 128), jnp.float32),
            pltpu.VMEM((SG * D,), jnp.float32),
            pltpu.VMEM((SG * D,), jnp.float32),
            pltpu.VMEM((BPW,), jnp.float32),
            pltpu.SemaphoreType.DMA,
        ],
    )
    return dot(uids, iids, ueT, ieT, ub1, ib1)


# speedup vs baseline: 1.2024x; 1.2024x over previous
import jax
import jax.numpy as jnp
from jax import lax
from jax.experimental import pallas as pl
from jax.experimental.pallas import tpu as pltpu
from jax.experimental.pallas import tpu_sc as plsc

B = 16384
D = 64
NW = 32
BPW = B // NW
L = 16
SG = 4


def _dot_body(uids, iids, uembT, iembT, out,
              uidx_v, iidx_v, slabs, ucol_v, icol_v, out_v, sem):
    wid = lax.axis_index("s") * 2 + lax.axis_index("c")
    base = wid * BPW

    pltpu.sync_copy(uids.at[pl.ds(base, BPW)], uidx_v)
    pltpu.sync_copy(iids.at[pl.ds(base, BPW)], iidx_v)

    dvec = lax.iota(jnp.int32, L)
    lane = lax.iota(jnp.int32, L)
    GIDS = 64
    NPH = 2 * (GIDS // SG)
    NBUF = 3

    def group(g, carry):
        idus = [uidx_v[pl.ds(g * GIDS + v * L, L)] for v in range(GIDS // L)]
        idis = [iidx_v[pl.ds(g * GIDS + v * L, L)] for v in range(GIDS // L)]

        def issue(p):
            q, ids = divmod(p, 2)
            idvecs = idis if ids else idus
            table = iembT if ids else uembT
            cps, offs = [], []
            for j in range(SG):
                e = q * SG + j
                tid = idvecs[e // L][e % L]
                c0 = pl.multiple_of((tid >> 7) << 7, 128)
                cps.append(pltpu.async_copy(
                    table.at[:, pl.ds(c0, 128)], slabs.at[p % NBUF, j], sem))
                offs.append(tid & 127)
            return cps, offs

        def extract(p, offs, col_v):
            for j in range(SG):
                o = jnp.full((L,), offs[j], jnp.int32)
                for k in range(D // L):
                    col_v[pl.ds(j * D + k * L, L)] = plsc.load_gather(
                        slabs.at[p % NBUF, j], [dvec + k * L, o])

        accvs = [jnp.zeros((L,), jnp.float32) for _ in range(GIDS // L)]
        ring = [issue(0), issue(1)]
        for p in range(NPH):
            if p + 2 < NPH:
                ring.append(issue(p + 2))
            cps, offs = ring[p]
            for c in cps:
                c.wait()
            q, ids = divmod(p, 2)
            extract(p, offs, icol_v if ids else ucol_v)
            if ids:
                for j in range(SG):
                    acc = (ucol_v[pl.ds(j * D, L)] * icol_v[pl.ds(j * D, L)])
                    for k in range(1, D // L):
                        acc = acc + (ucol_v[pl.ds(j * D + k * L, L)]
                                     * icol_v[pl.ds(j * D + k * L, L)])
                    s = jnp.sum(acc)
                    e = q * SG + j
                    accvs[e // L] = jnp.where(
                        lane == e % L, s, accvs[e // L])
        for v in range(GIDS // L):
            out_v[pl.ds(g * GIDS + v * L, L)] = accvs[v]
        return carry

    lax.fori_loop(0, BPW // GIDS, group, 0)
    pltpu.sync_copy(out_v, out.at[pl.ds(base, BPW)])


def _bias_body(uids, iids, ubias, ibias, partial, out,
               uidx_v, iidx_v, ub_v, ib_v, p_v, sem):
    wid = lax.axis_index("s") * 2 + lax.axis_index("c")
    base = wid * BPW

    pltpu.sync_copy(uids.at[pl.ds(base, BPW)], uidx_v)
    pltpu.sync_copy(iids.at[pl.ds(base, BPW)], iidx_v)
    pltpu.sync_copy(partial.at[pl.ds(base, BPW)], p_v)
    c1 = pltpu.async_copy(ubias.at[uidx_v], ub_v, sem)
    c2 = pltpu.async_copy(ibias.at[iidx_v], ib_v, sem)
    c1.wait()
    c2.wait()

    def group(g, carry):
        s = pl.ds(g * L, L)
        p_v[s] = p_v[s] + ub_v[s] + ib_v[s]
        return carry

    lax.fori_loop(0, BPW // L, group, 0)
    pltpu.sync_copy(p_v, out.at[pl.ds(base, BPW)])


def kernel(user_ids, item_ids, user_embed, item_embed, user_bias, item_bias):
    uids = user_ids.astype(jnp.int32)
    iids = item_ids.astype(jnp.int32)
    ueT = user_embed.T
    ieT = item_embed.T
    ub1 = user_bias.reshape(-1)
    ib1 = item_bias.reshape(-1)

    mesh = plsc.VectorSubcoreMesh(core_axis_name="c", subcore_axis_name="s")
    dot = pl.kernel(
        _dot_body,
        mesh=mesh,
        out_type=jax.ShapeDtypeStruct((B,), jnp.float32),
        compiler_params=pltpu.CompilerParams(
            needs_layout_passes=False, use_tc_tiling_on_sc=True
        ),
        scratch_types=[
            pltpu.VMEM((BPW,), jnp.int32),
            pltpu.VMEM((BPW,), jnp.int32),
            pltpu.VMEM((3, SG, D, 128), jnp.float32),
            pltpu.VMEM((SG * D,), jnp.float32),
            pltpu.VMEM((SG * D,), jnp.float32),
            pltpu.VMEM((BPW,), jnp.float32),
            pltpu.SemaphoreType.DMA,
        ],
    )
    partial = dot(uids, iids, ueT, ieT)

    biased = pl.kernel(
        _bias_body,
        mesh=mesh,
        out_type=jax.ShapeDtypeStruct((B,), jnp.float32),
        compiler_params=pltpu.CompilerParams(
            needs_layout_passes=False, use_tc_tiling_on_sc=False
        ),
        scratch_types=[
            pltpu.VMEM((BPW,), jnp.int32),
            pltpu.VMEM((BPW,), jnp.int32),
            pltpu.VMEM((BPW,), jnp.float32),
            pltpu.VMEM((BPW,), jnp.float32),
            pltpu.VMEM((BPW,), jnp.float32),
            pltpu.SemaphoreType.DMA,
        ],
    )
    return biased(uids, iids, ub1, ib1, partial)
